# Initial kernel scaffold; baseline (speedup 1.0000x reference)
import jax
import jax.numpy as jnp
from jax.experimental import pallas as pl


def kernel(h_trans, Wq, bq, Wk, bk, Wv, bv):
    raise NotImplementedError("write your pallas kernel here")


# speedup vs baseline: 2.6620x; 2.6620x over previous
import jax
import jax.numpy as jnp
from jax import lax
from jax.experimental import pallas as pl

N = 100000
D = 128
H = 4
DH = 32

_B1 = 5000
_B2 = 5000
_PREC = lax.Precision.HIGHEST


def _stats_body(h_ref, c_ref, cs_ref):
    i = pl.program_id(0)
    h = h_ref[...]
    c = lax.dot_general(h, h, (((0,), (0,)), ((), ())),
                        preferred_element_type=jnp.float32, precision=_PREC)
    cs = jnp.broadcast_to(jnp.sum(h, axis=0, keepdims=True), (8, D))

    @pl.when(i == 0)
    def _init():
        c_ref[...] = c
        cs_ref[...] = cs

    @pl.when(i != 0)
    def _acc():
        c_ref[...] += c
        cs_ref[...] += cs


def _derive_body(c_ref, cs_ref, wq_ref, bq_ref, wk_ref, bk_ref, wv_ref, bv_ref,
                 pcat_ref, rcat_ref):
    C = c_ref[...]
    cs = cs_ref[0:1, :]
    Wq, Wk, Wv = wq_ref[...], wk_ref[...], wv_ref[...]
    bq, bk, bv = bq_ref[0:1, :], bk_ref[0:1, :], bv_ref[0:1, :]

    row = lax.broadcasted_iota(jnp.int32, (D, D), 0)
    col = lax.broadcasted_iota(jnp.int32, (D, D), 1)
    eye = (row == col).astype(jnp.float32)
    mask = ((row // DH) == (col // DH)).astype(jnp.float32)

    nd = jnp.float32(N * D)
    s1 = jnp.sum(cs)
    s2 = jnp.sum(C * eye)
    mu = s1 / nd
    var = s2 / nd - mu * mu
    alpha = lax.rsqrt(var + 1e-5)

    def dotg(a, b, dims):
        return lax.dot_general(a, b, (dims, ((), ())),
                               preferred_element_type=jnp.float32,
                               precision=_PREC)

    c_q = bq - mu * alpha * jnp.sum(Wq, axis=1)[None, :]
    c_k = bk - mu * alpha * jnp.sum(Wk, axis=1)[None, :]
    c_v = bv - mu * alpha * jnp.sum(Wv, axis=1)[None, :]

    u_q = dotg(cs, Wq, ((1,), (1,)))
    u_k = dotg(cs, Wk, ((1,), (1,)))
    u_v = dotg(cs, Wv, ((1,), (1,)))

    WqC = dotg(Wq, C, ((1,), (0,)))
    WkC = dotg(Wk, C, ((1,), (0,)))
    tr_q = jnp.sum(WqC * Wq)
    tr_k = jnp.sum(WkC * Wk)
    s_q = alpha * alpha * tr_q + 2.0 * alpha * jnp.sum(u_q * c_q) \
        + N * jnp.sum(c_q * c_q)
    s_k = alpha * alpha * tr_k + 2.0 * alpha * jnp.sum(u_k * c_k) \
        + N * jnp.sum(c_k * c_k)
    inv_nqk = lax.rsqrt(s_q) * lax.rsqrt(s_k)

    def outer(a, b):
        return dotg(a, b, ((0,), (0,)))

    KV = (alpha * alpha) * dotg(WkC, Wv, ((1,), (1,))) \
        + alpha * outer(u_k, c_v) + alpha * outer(c_k, u_v) \
        + jnp.float32(N) * outer(c_k, c_v)
    BD = KV * mask

    ksum = alpha * u_k + jnp.float32(N) * c_k
    ones = jnp.ones((1, D), jnp.float32)
    G = outer(ksum, ones) * mask

    WvT = dotg(eye, Wv, ((1,), (1,)))

    Pn = alpha * inv_nqk * dotg(Wq, BD, ((0,), (0,))) \
        + jnp.float32(N) * alpha * WvT
    rn = inv_nqk * dotg(c_q, BD, ((1,), (0,))) + jnp.float32(N) * c_v
    Pd = alpha * inv_nqk * dotg(Wq, G, ((0,), (0,)))
    rd = inv_nqk * dotg(c_q, G, ((1,), (0,))) + jnp.float32(N)

    pcat_ref[...] = jnp.concatenate([Pn, Pd], axis=1)
    rcat_ref[...] = jnp.broadcast_to(
        jnp.concatenate([rn, rd], axis=1), (8, 2 * D))


def _apply_body(h_ref, pcat_ref, rcat_ref, out_ref):
    h = h_ref[...]
    r = lax.dot_general(h, pcat_ref[...], (((1,), (0,)), ((), ())),
                        preferred_element_type=jnp.float32, precision=_PREC)
    num = r[:, :D] + rcat_ref[0:1, :D]
    den = r[:, D:] + rcat_ref[0:1, D:]
    out_ref[...] = num / den


def kernel(h_trans, Wq, bq, Wk, bk, Wv, bv):
    n = h_trans.shape[0]

    C, cs = pl.pallas_call(
        _stats_body,
        grid=(n // _B1,),
        in_specs=[pl.BlockSpec((_B1, D), lambda i: (i, 0))],
        out_specs=[pl.BlockSpec((D, D), lambda i: (0, 0)),
                   pl.BlockSpec((8, D), lambda i: (0, 0))],
        out_shape=[jax.ShapeDtypeStruct((D, D), jnp.float32),
                   jax.ShapeDtypeStruct((8, D), jnp.float32)],
    )(h_trans)

    bq2, bk2, bv2 = (jnp.broadcast_to(b[None, :], (8, D)) for b in (bq, bk, bv))
    Pcat, rcat = pl.pallas_call(
        _derive_body,
        out_shape=[jax.ShapeDtypeStruct((D, 2 * D), jnp.float32),
                   jax.ShapeDtypeStruct((8, 2 * D), jnp.float32)],
    )(C, cs, Wq, bq2, Wk, bk2, Wv, bv2)

    out = pl.pallas_call(
        _apply_body,
        grid=(n // _B2,),
        in_specs=[pl.BlockSpec((_B2, D), lambda i: (i, 0)),
                  pl.BlockSpec((D, 2 * D), lambda i: (0, 0)),
                  pl.BlockSpec((8, 2 * D), lambda i: (0, 0))],
        out_specs=pl.BlockSpec((_B2, D), lambda i: (i, 0)),
        out_shape=jax.ShapeDtypeStruct((n, D), jnp.float32),
    )(h_trans, Pcat, rcat)
    return out


# speedup vs baseline: 5.5260x; 2.0759x over previous
import jax
import jax.numpy as jnp
from jax import lax
from jax.experimental import pallas as pl

N = 100000
D = 128
H = 4
DH = 32

_B1 = 5000
_B2 = 5000
_PREC = lax.Precision.DEFAULT


def _stats_body(h_ref, c_ref, cs_ref):
    i = pl.program_id(0)
    h = h_ref[...]
    c = lax.dot_general(h, h, (((0,), (0,)), ((), ())),
                        preferred_element_type=jnp.float32, precision=_PREC)
    cs = jnp.broadcast_to(jnp.sum(h, axis=0, keepdims=True), (8, D))

    @pl.when(i == 0)
    def _init():
        c_ref[...] = c
        cs_ref[...] = cs

    @pl.when(i != 0)
    def _acc():
        c_ref[...] += c
        cs_ref[...] += cs


def _derive_body(c_ref, cs_ref, wq_ref, bq_ref, wk_ref, bk_ref, wv_ref, bv_ref,
                 pcat_ref, rcat_ref):
    C = c_ref[...]
    cs = cs_ref[0:1, :]
    Wq, Wk, Wv = wq_ref[...], wk_ref[...], wv_ref[...]
    bq, bk, bv = bq_ref[0:1, :], bk_ref[0:1, :], bv_ref[0:1, :]

    row = lax.broadcasted_iota(jnp.int32, (D, D), 0)
    col = lax.broadcasted_iota(jnp.int32, (D, D), 1)
    eye = (row == col).astype(jnp.float32)
    mask = ((row // DH) == (col // DH)).astype(jnp.float32)

    nd = jnp.float32(N * D)
    s1 = jnp.sum(cs)
    s2 = jnp.sum(C * eye)
    mu = s1 / nd
    var = s2 / nd - mu * mu
    alpha = lax.rsqrt(var + 1e-5)

    def dotg(a, b, dims):
        return lax.dot_general(a, b, (dims, ((), ())),
                               preferred_element_type=jnp.float32,
                               precision=_PREC)

    c_q = bq - mu * alpha * jnp.sum(Wq, axis=1)[None, :]
    c_k = bk - mu * alpha * jnp.sum(Wk, axis=1)[None, :]
    c_v = bv - mu * alpha * jnp.sum(Wv, axis=1)[None, :]

    u_q = dotg(cs, Wq, ((1,), (1,)))
    u_k = dotg(cs, Wk, ((1,), (1,)))
    u_v = dotg(cs, Wv, ((1,), (1,)))

    WqC = dotg(Wq, C, ((1,), (0,)))
    WkC = dotg(Wk, C, ((1,), (0,)))
    tr_q = jnp.sum(WqC * Wq)
    tr_k = jnp.sum(WkC * Wk)
    s_q = alpha * alpha * tr_q + 2.0 * alpha * jnp.sum(u_q * c_q) \
        + N * jnp.sum(c_q * c_q)
    s_k = alpha * alpha * tr_k + 2.0 * alpha * jnp.sum(u_k * c_k) \
        + N * jnp.sum(c_k * c_k)
    inv_nqk = lax.rsqrt(s_q) * lax.rsqrt(s_k)

    def outer(a, b):
        return dotg(a, b, ((0,), (0,)))

    KV = (alpha * alpha) * dotg(WkC, Wv, ((1,), (1,))) \
        + alpha * outer(u_k, c_v) + alpha * outer(c_k, u_v) \
        + jnp.float32(N) * outer(c_k, c_v)
    BD = KV * mask

    ksum = alpha * u_k + jnp.float32(N) * c_k
    ones = jnp.ones((1, D), jnp.float32)
    G = outer(ksum, ones) * mask

    WvT = dotg(eye, Wv, ((1,), (1,)))

    Pn = alpha * inv_nqk * dotg(Wq, BD, ((0,), (0,))) \
        + jnp.float32(N) * alpha * WvT
    rn = inv_nqk * dotg(c_q, BD, ((1,), (0,))) + jnp.float32(N) * c_v
    Pd = alpha * inv_nqk * dotg(Wq, G, ((0,), (0,)))
    rd = inv_nqk * dotg(c_q, G, ((1,), (0,))) + jnp.float32(N)

    pcat_ref[...] = jnp.concatenate([Pn, Pd], axis=1)
    rcat_ref[...] = jnp.broadcast_to(
        jnp.concatenate([rn, rd], axis=1), (8, 2 * D))


def _apply_body(h_ref, pcat_ref, rcat_ref, out_ref):
    h = h_ref[...]
    r = lax.dot_general(h, pcat_ref[...], (((1,), (0,)), ((), ())),
                        preferred_element_type=jnp.float32, precision=_PREC)
    num = r[:, :D] + rcat_ref[0:1, :D]
    den = r[:, D:] + rcat_ref[0:1, D:]
    out_ref[...] = num / den


def kernel(h_trans, Wq, bq, Wk, bk, Wv, bv):
    n = h_trans.shape[0]

    C, cs = pl.pallas_call(
        _stats_body,
        grid=(n // _B1,),
        in_specs=[pl.BlockSpec((_B1, D), lambda i: (i, 0))],
        out_specs=[pl.BlockSpec((D, D), lambda i: (0, 0)),
                   pl.BlockSpec((8, D), lambda i: (0, 0))],
        out_shape=[jax.ShapeDtypeStruct((D, D), jnp.float32),
                   jax.ShapeDtypeStruct((8, D), jnp.float32)],
    )(h_trans)

    bq2, bk2, bv2 = (jnp.broadcast_to(b[None, :], (8, D)) for b in (bq, bk, bv))
    Pcat, rcat = pl.pallas_call(
        _derive_body,
        out_shape=[jax.ShapeDtypeStruct((D, 2 * D), jnp.float32),
                   jax.ShapeDtypeStruct((8, 2 * D), jnp.float32)],
    )(C, cs, Wq, bq2, Wk, bk2, Wv, bv2)

    out = pl.pallas_call(
        _apply_body,
        grid=(n // _B2,),
        in_specs=[pl.BlockSpec((_B2, D), lambda i: (i, 0)),
                  pl.BlockSpec((D, 2 * D), lambda i: (0, 0)),
                  pl.BlockSpec((8, 2 * D), lambda i: (0, 0))],
        out_specs=pl.BlockSpec((_B2, D), lambda i: (i, 0)),
        out_shape=jax.ShapeDtypeStruct((n, D), jnp.float32),
    )(h_trans, Pcat, rcat)
    return out


# speedup vs baseline: 6.3474x; 1.1486x over previous
import jax
import jax.numpy as jnp
from jax import lax
from jax.experimental import pallas as pl

N = 100000
D = 128
H = 4
DH = 32

_B1 = 10000
_B2 = 10000
_PREC = lax.Precision.DEFAULT


def _stats_body(h_ref, c_ref, cs_ref):
    i = pl.program_id(0)
    h = h_ref[...]
    c = lax.dot_general(h, h, (((0,), (0,)), ((), ())),
                        preferred_element_type=jnp.float32, precision=_PREC)
    cs = jnp.broadcast_to(jnp.sum(h, axis=0, keepdims=True), (8, D))

    @pl.when(i == 0)
    def _init():
        c_ref[...] = c
        cs_ref[...] = cs

    @pl.when(i != 0)
    def _acc():
        c_ref[...] += c
        cs_ref[...] += cs


def _derive_body(c_ref, cs_ref, wq_ref, bq_ref, wk_ref, bk_ref, wv_ref, bv_ref,
                 pcat_ref, rcat_ref):
    C = c_ref[...]
    cs = cs_ref[0:1, :]
    Wq, Wk, Wv = wq_ref[...], wk_ref[...], wv_ref[...]
    bq, bk, bv = bq_ref[0:1, :], bk_ref[0:1, :], bv_ref[0:1, :]

    row = lax.broadcasted_iota(jnp.int32, (D, D), 0)
    col = lax.broadcasted_iota(jnp.int32, (D, D), 1)
    eye = (row == col).astype(jnp.float32)
    mask = ((row // DH) == (col // DH)).astype(jnp.float32)

    nd = jnp.float32(N * D)
    s1 = jnp.sum(cs)
    s2 = jnp.sum(C * eye)
    mu = s1 / nd
    var = s2 / nd - mu * mu
    alpha = lax.rsqrt(var + 1e-5)

    def dotg(a, b, dims):
        return lax.dot_general(a, b, (dims, ((), ())),
                               preferred_element_type=jnp.float32,
                               precision=_PREC)

    c_q = bq - mu * alpha * jnp.sum(Wq, axis=1)[None, :]
    c_k = bk - mu * alpha * jnp.sum(Wk, axis=1)[None, :]
    c_v = bv - mu * alpha * jnp.sum(Wv, axis=1)[None, :]

    u_q = dotg(cs, Wq, ((1,), (1,)))
    u_k = dotg(cs, Wk, ((1,), (1,)))
    u_v = dotg(cs, Wv, ((1,), (1,)))

    WqC = dotg(Wq, C, ((1,), (0,)))
    WkC = dotg(Wk, C, ((1,), (0,)))
    tr_q = jnp.sum(WqC * Wq)
    tr_k = jnp.sum(WkC * Wk)
    s_q = alpha * alpha * tr_q + 2.0 * alpha * jnp.sum(u_q * c_q) \
        + N * jnp.sum(c_q * c_q)
    s_k = alpha * alpha * tr_k + 2.0 * alpha * jnp.sum(u_k * c_k) \
        + N * jnp.sum(c_k * c_k)
    inv_nqk = lax.rsqrt(s_q) * lax.rsqrt(s_k)

    def outer(a, b):
        return dotg(a, b, ((0,), (0,)))

    KV = (alpha * alpha) * dotg(WkC, Wv, ((1,), (1,))) \
        + alpha * outer(u_k, c_v) + alpha * outer(c_k, u_v) \
        + jnp.float32(N) * outer(c_k, c_v)
    BD = KV * mask

    ksum = alpha * u_k + jnp.float32(N) * c_k
    ones = jnp.ones((1, D), jnp.float32)
    G = outer(ksum, ones) * mask

    WvT = dotg(eye, Wv, ((1,), (1,)))

    Pn = alpha * inv_nqk * dotg(Wq, BD, ((0,), (0,))) \
        + jnp.float32(N) * alpha * WvT
    rn = inv_nqk * dotg(c_q, BD, ((1,), (0,))) + jnp.float32(N) * c_v
    Pd = alpha * inv_nqk * dotg(Wq, G, ((0,), (0,)))
    rd = inv_nqk * dotg(c_q, G, ((1,), (0,))) + jnp.float32(N)

    pcat_ref[...] = jnp.concatenate([Pn, Pd], axis=1)
    rcat_ref[...] = jnp.broadcast_to(
        jnp.concatenate([rn, rd], axis=1), (8, 2 * D))


def _apply_body(h_ref, pcat_ref, rcat_ref, out_ref):
    h = h_ref[...]
    r = lax.dot_general(h, pcat_ref[...], (((1,), (0,)), ((), ())),
                        preferred_element_type=jnp.float32, precision=_PREC)
    num = r[:, :D] + rcat_ref[0:1, :D]
    den = r[:, D:] + rcat_ref[0:1, D:]
    out_ref[...] = num / den


def kernel(h_trans, Wq, bq, Wk, bk, Wv, bv):
    n = h_trans.shape[0]

    C, cs = pl.pallas_call(
        _stats_body,
        grid=(n // _B1,),
        in_specs=[pl.BlockSpec((_B1, D), lambda i: (i, 0))],
        out_specs=[pl.BlockSpec((D, D), lambda i: (0, 0)),
                   pl.BlockSpec((8, D), lambda i: (0, 0))],
        out_shape=[jax.ShapeDtypeStruct((D, D), jnp.float32),
                   jax.ShapeDtypeStruct((8, D), jnp.float32)],
    )(h_trans)

    bq2, bk2, bv2 = (jnp.broadcast_to(b[None, :], (8, D)) for b in (bq, bk, bv))
    Pcat, rcat = pl.pallas_call(
        _derive_body,
        out_shape=[jax.ShapeDtypeStruct((D, 2 * D), jnp.float32),
                   jax.ShapeDtypeStruct((8, 2 * D), jnp.float32)],
    )(C, cs, Wq, bq2, Wk, bk2, Wv, bv2)

    out = pl.pallas_call(
        _apply_body,
        grid=(n // _B2,),
        in_specs=[pl.BlockSpec((_B2, D), lambda i: (i, 0)),
                  pl.BlockSpec((D, 2 * D), lambda i: (0, 0)),
                  pl.BlockSpec((8, 2 * D), lambda i: (0, 0))],
        out_specs=pl.BlockSpec((_B2, D), lambda i: (i, 0)),
        out_shape=jax.ShapeDtypeStruct((n, D), jnp.float32),
    )(h_trans, Pcat, rcat)
    return out


# speedup vs baseline: 8.4517x; 1.3315x over previous
import functools
import jax
import jax.numpy as jnp
from jax import lax
from jax.experimental import pallas as pl
from jax.experimental.pallas import tpu as pltpu

N = 100000
D = 128
H = 4
DH = 32

_B = 10000
_NB = N // _B
_PREC = lax.Precision.DEFAULT


def _derive(C, cs, Wq, bq, Wk, bk, Wv, bv):
    row = lax.broadcasted_iota(jnp.int32, (D, D), 0)
    col = lax.broadcasted_iota(jnp.int32, (D, D), 1)
    eye = (row == col).astype(jnp.float32)
    mask = ((row // DH) == (col // DH)).astype(jnp.float32)

    nd = jnp.float32(N * D)
    s1 = jnp.sum(cs)
    s2 = jnp.sum(C * eye)
    mu = s1 / nd
    var = s2 / nd - mu * mu
    alpha = lax.rsqrt(var + 1e-5)

    def dotg(a, b, dims):
        return lax.dot_general(a, b, (dims, ((), ())),
                               preferred_element_type=jnp.float32,
                               precision=_PREC)

    c_q = bq - mu * alpha * jnp.sum(Wq, axis=1)[None, :]
    c_k = bk - mu * alpha * jnp.sum(Wk, axis=1)[None, :]
    c_v = bv - mu * alpha * jnp.sum(Wv, axis=1)[None, :]

    u_q = dotg(cs, Wq, ((1,), (1,)))
    u_k = dotg(cs, Wk, ((1,), (1,)))
    u_v = dotg(cs, Wv, ((1,), (1,)))

    WqC = dotg(Wq, C, ((1,), (0,)))
    WkC = dotg(Wk, C, ((1,), (0,)))
    s_q = alpha * alpha * jnp.sum(WqC * Wq) \
        + 2.0 * alpha * jnp.sum(u_q * c_q) + N * jnp.sum(c_q * c_q)
    s_k = alpha * alpha * jnp.sum(WkC * Wk) \
        + 2.0 * alpha * jnp.sum(u_k * c_k) + N * jnp.sum(c_k * c_k)
    inv_nqk = lax.rsqrt(s_q) * lax.rsqrt(s_k)

    def outer(a, b):
        return dotg(a, b, ((0,), (0,)))

    KV = (alpha * alpha) * dotg(WkC, Wv, ((1,), (1,))) \
        + alpha * outer(u_k, c_v) + alpha * outer(c_k, u_v) \
        + jnp.float32(N) * outer(c_k, c_v)
    BD = KV * mask

    ksum = alpha * u_k + jnp.float32(N) * c_k
    ones = jnp.ones((1, D), jnp.float32)
    G = outer(ksum, ones) * mask

    WvT = dotg(eye, Wv, ((1,), (1,)))

    Pn = alpha * inv_nqk * dotg(Wq, BD, ((0,), (0,))) \
        + jnp.float32(N) * alpha * WvT
    rn = inv_nqk * dotg(c_q, BD, ((1,), (0,))) + jnp.float32(N) * c_v
    Pd = alpha * inv_nqk * dotg(Wq, G, ((0,), (0,)))
    rd = inv_nqk * dotg(c_q, G, ((1,), (0,))) + jnp.float32(N)

    return jnp.concatenate([Pn, Pd], axis=1), jnp.concatenate([rn, rd], axis=1)


def _fused_body(h_ref, wq_ref, bq_ref, wk_ref, bk_ref, wv_ref, bv_ref,
                out_ref, hc_ref, c_ref, cs_ref, pcat_ref, rcat_ref):
    p = pl.program_id(0)
    j = pl.program_id(1)

    @pl.when(p == 0)
    def _phase0():
        h = h_ref[...]
        hc_ref[j] = h.astype(jnp.bfloat16)
        c = lax.dot_general(h, h, (((0,), (0,)), ((), ())),
                            preferred_element_type=jnp.float32,
                            precision=_PREC)
        cs = jnp.sum(h, axis=0, keepdims=True)

        @pl.when(j == 0)
        def _init():
            c_ref[...] = c
            cs_ref[...] = cs

        @pl.when(j != 0)
        def _acc():
            c_ref[...] += c
            cs_ref[...] += cs

    @pl.when((p == 0) & (j == _NB - 1))
    def _derive_step():
        pcat, rcat = _derive(c_ref[...], cs_ref[...],
                             wq_ref[...], bq_ref[0:1, :],
                             wk_ref[...], bk_ref[0:1, :],
                             wv_ref[...], bv_ref[0:1, :])
        pcat_ref[...] = pcat.astype(jnp.bfloat16)
        rcat_ref[...] = rcat

    @pl.when(p == 1)
    def _phase1():
        hb = hc_ref[j]
        r = lax.dot_general(hb, pcat_ref[...], (((1,), (0,)), ((), ())),
                            preferred_element_type=jnp.float32,
                            precision=_PREC)
        num = r[:, :D] + rcat_ref[0:1, :D]
        den = r[:, D:] + rcat_ref[0:1, D:]
        out_ref[...] = num / den


def kernel(h_trans, Wq, bq, Wk, bk, Wv, bv):
    n = h_trans.shape[0]
    nb = n // _B
    b8 = (jnp.broadcast_to(b[None, :], (8, D)) for b in (bq, bk, bv))
    bq2, bk2, bv2 = b8
    return pl.pallas_call(
        _fused_body,
        grid=(2, nb),
        in_specs=[
            pl.BlockSpec((_B, D), lambda p, j: (j * (1 - p), 0)),
            pl.BlockSpec((D, D), lambda p, j: (0, 0)),
            pl.BlockSpec((8, D), lambda p, j: (0, 0)),
            pl.BlockSpec((D, D), lambda p, j: (0, 0)),
            pl.BlockSpec((8, D), lambda p, j: (0, 0)),
            pl.BlockSpec((D, D), lambda p, j: (0, 0)),
            pl.BlockSpec((8, D), lambda p, j: (0, 0)),
        ],
        out_specs=pl.BlockSpec((_B, D), lambda p, j: (j * p, 0)),
        out_shape=jax.ShapeDtypeStruct((n, D), jnp.float32),
        scratch_shapes=[
            pltpu.VMEM((_NB, _B, D), jnp.bfloat16),
            pltpu.VMEM((D, D), jnp.float32),
            pltpu.VMEM((1, D), jnp.float32),
            pltpu.VMEM((D, 2 * D), jnp.bfloat16),
            pltpu.VMEM((1, 2 * D), jnp.float32),
        ],
        compiler_params=pltpu.CompilerParams(
            vmem_limit_bytes=100 * 1024 * 1024,
            dimension_semantics=("arbitrary", "arbitrary"),
        ),
    )(h_trans, Wq, bq2, Wk, bk2, Wv, bv2)
